# baseline (speedup 1.0000x reference)
import functools

import jax
import jax.numpy as jnp
from jax import lax
from jax.experimental import pallas as pl
from jax.experimental.pallas import tpu as pltpu
from jax.experimental.pallas import tpu_sc as plsc

B = 4096
L = 50
LP = 56
D = 64
DS = 16

NC, NS = 2, 16
NW = NC * NS
ROWS_PER_W = B * LP // NW
SUB = 128
K_INNER = 8
CHUNK = SUB * K_INNER
N_OUTER = ROWS_PER_W // CHUNK
N_IDX_ROWS = ROWS_PER_W // SUB
ACC_PER_W = B // NW


def _make_sc_gather(interpret=False):
    mesh = plsc.VectorSubcoreMesh(core_axis_name="c", subcore_axis_name="s",
                                  num_cores=NC, num_subcores=NS)

    @functools.partial(
        pl.kernel,
        out_type=(
            jax.ShapeDtypeStruct((B * LP, D), jnp.float32),
            jax.ShapeDtypeStruct((B, D), jnp.float32),
            jax.ShapeDtypeStruct((B, DS), jnp.float32),
            jax.ShapeDtypeStruct((B, DS), jnp.float32),
        ),
        mesh=mesh,
        scratch_types=[
            pltpu.VMEM((N_IDX_ROWS, SUB), jnp.int32),
            pltpu.VMEM((CHUNK, D), jnp.float32),
            pltpu.VMEM((1, ACC_PER_W), jnp.int32),
            pltpu.VMEM((ACC_PER_W, D), jnp.float32),
            pltpu.VMEM((1, ACC_PER_W), jnp.int32),
            pltpu.VMEM((ACC_PER_W, DS), jnp.float32),
            pltpu.VMEM((1, ACC_PER_W), jnp.int32),
            pltpu.VMEM((ACC_PER_W, DS), jnp.float32),
            pltpu.SemaphoreType.DMA,
            pltpu.SemaphoreType.DMA,
        ],
        compiler_params=pltpu.CompilerParams(use_tc_tiling_on_sc=False),
        interpret=interpret,
    )
    def sc_gather(chain_ids2d, acc_idx2d, hour_idx2d, wd_idx2d,
                  chain_table, account_table, hour_table, weekday_table,
                  x_out, acc_out, h_out, w_out,
                  cidx_v, rows_v, aidx_v, arows_v, hidx_v, hrows_v,
                  widx_v, wrows_v, sem, sem2):
        wid = lax.axis_index("s") * NC + lax.axis_index("c")

        pltpu.sync_copy(acc_idx2d.at[wid], aidx_v)
        pltpu.sync_copy(hour_idx2d.at[wid], hidx_v)
        pltpu.sync_copy(wd_idx2d.at[wid], widx_v)
        a_cp = pltpu.make_async_copy(account_table.at[aidx_v.at[0]], arows_v,
                                     sem2)
        h_cp = pltpu.make_async_copy(hour_table.at[hidx_v.at[0]], hrows_v,
                                     sem2)
        w_cp = pltpu.make_async_copy(weekday_table.at[widx_v.at[0]], wrows_v,
                                     sem2)
        a_cp.start()
        h_cp.start()
        w_cp.start()

        pltpu.sync_copy(chain_ids2d.at[wid], cidx_v)
        base = wid * ROWS_PER_W

        def outer(g, carry):
            cps = [
                pltpu.make_async_copy(
                    chain_table.at[cidx_v.at[g * K_INNER + j]],
                    rows_v.at[pl.ds(j * SUB, SUB)], sem)
                for j in range(K_INNER)
            ]
            for cp in cps:
                cp.start()
            for cp in cps:
                cp.wait()
            pltpu.sync_copy(rows_v, x_out.at[pl.ds(base + g * CHUNK, CHUNK)])
            return carry

        lax.fori_loop(0, N_OUTER, outer, 0)

        a_cp.wait()
        h_cp.wait()
        w_cp.wait()
        abase = wid * ACC_PER_W
        pltpu.sync_copy(arows_v, acc_out.at[pl.ds(abase, ACC_PER_W)])
        pltpu.sync_copy(hrows_v, h_out.at[pl.ds(abase, ACC_PER_W)])
        pltpu.sync_copy(wrows_v, w_out.at[pl.ds(abase, ACC_PER_W)])

    return sc_gather


BB = 128
PREC = lax.Precision.DEFAULT


def _attn_body(ids_ref, x_ref, acc_ref, h_ref, w_ref,
               wq_ref, bq_ref, wk_ref, bk_ref, wv_ref, bv_ref,
               wo_ref, bo_ref, lng_ref, lnb_ref, out_ref):
    x = x_ref[...]
    x2 = x.reshape(BB * LP, D)
    q = jnp.dot(x2, wq_ref[...], precision=PREC,
                preferred_element_type=jnp.float32) + bq_ref[...]
    k = jnp.dot(x2, wk_ref[...], precision=PREC,
                preferred_element_type=jnp.float32) + bk_ref[...]
    v = jnp.dot(x2, wv_ref[...], precision=PREC,
                preferred_element_type=jnp.float32) + bv_ref[...]
    q3 = q.reshape(BB, LP, D)
    k3 = k.reshape(BB, LP, D)
    v3 = v.reshape(BB, LP, D)
    scores = lax.dot_general(q3, k3, (((2,), (2,)), ((0,), (0,))),
                             precision=PREC,
                             preferred_element_type=jnp.float32) * 0.125
    mask = ids_ref[...] != 0
    scores = jnp.where(mask[:, None, :], scores, jnp.float32(-1e9))
    m = jnp.max(scores, axis=-1, keepdims=True)
    e = jnp.exp(scores - m)
    attn = e / jnp.sum(e, axis=-1, keepdims=True)
    ctx = lax.dot_general(attn, v3, (((2,), (1,)), ((0,), (0,))),
                          precision=PREC,
                          preferred_element_type=jnp.float32)
    ctx2 = ctx.reshape(BB * LP, D)
    y = x2 + jnp.dot(ctx2, wo_ref[...], precision=PREC,
                     preferred_element_type=jnp.float32) + bo_ref[...]
    mean = jnp.mean(y, axis=-1, keepdims=True)
    var = jnp.mean((y - mean) ** 2, axis=-1, keepdims=True)
    y = lng_ref[...] * (y - mean) / jnp.sqrt(var + 1e-3) + lnb_ref[...]
    y3 = y.reshape(BB, LP, D)
    mf = mask.astype(jnp.float32)[..., None]
    pooled = jnp.sum(y3 * mf, axis=1) / jnp.clip(jnp.sum(mf, axis=1), 1.0,
                                                 None)
    out_ref[:, 0:D] = acc_ref[...]
    out_ref[:, D:2 * D] = pooled
    out_ref[:, 2 * D:2 * D + DS] = h_ref[...]
    out_ref[:, 2 * D + DS:2 * D + 2 * DS] = w_ref[...]


def _make_tc_attn(interpret=False):
    full = lambda shape: pl.BlockSpec(shape, lambda i: (0,) * len(shape))
    return pl.pallas_call(
        _attn_body,
        grid=(B // BB,),
        in_specs=[
            pl.BlockSpec((BB, LP), lambda i: (i, 0)),
            pl.BlockSpec((BB, LP, D), lambda i: (i, 0, 0)),
            pl.BlockSpec((BB, D), lambda i: (i, 0)),
            pl.BlockSpec((BB, DS), lambda i: (i, 0)),
            pl.BlockSpec((BB, DS), lambda i: (i, 0)),
            full((D, D)), full((1, D)),
            full((D, D)), full((1, D)),
            full((D, D)), full((1, D)),
            full((D, D)), full((1, D)),
            full((1, D)), full((1, D)),
        ],
        out_specs=pl.BlockSpec((BB, 2 * D + 2 * DS), lambda i: (i, 0)),
        out_shape=jax.ShapeDtypeStruct((B, 2 * D + 2 * DS), jnp.float32),
        interpret=interpret,
    )


_tc_attn = _make_tc_attn()


@functools.cache
def _get_sc_gather():
    return _make_sc_gather()


def kernel(account_idx, prev_chain_ids, hour_idx, weekday_idx,
           account_table, chain_table, Wq, bq, Wk, bk, Wv, bv, Wo, bo,
           ln_gamma, ln_beta, hour_table, weekday_table):
    ids_pad = jnp.concatenate(
        [prev_chain_ids,
         jnp.zeros((B, LP - L), dtype=prev_chain_ids.dtype)], axis=1)
    chain_ids2d = ids_pad.reshape(NW, N_IDX_ROWS, SUB)
    acc_idx2d = account_idx.reshape(NW, 1, ACC_PER_W)
    hour_idx2d = hour_idx.reshape(NW, 1, ACC_PER_W)
    wd_idx2d = weekday_idx.reshape(NW, 1, ACC_PER_W)
    xf, acc, h, w = _get_sc_gather()(chain_ids2d, acc_idx2d, hour_idx2d,
                                     wd_idx2d, chain_table, account_table,
                                     hour_table, weekday_table)
    return _tc_attn(ids_pad, xf.reshape(B, LP, D), acc, h, w,
                    Wq, bq.reshape(1, D), Wk, bk.reshape(1, D),
                    Wv, bv.reshape(1, D), Wo, bo.reshape(1, D),
                    ln_gamma.reshape(1, D), ln_beta.reshape(1, D))


# speedup vs baseline: 1.9601x; 1.9601x over previous
import functools

import jax
import jax.numpy as jnp
from jax import lax
from jax.experimental import pallas as pl
from jax.experimental.pallas import tpu as pltpu
from jax.experimental.pallas import tpu_sc as plsc

B = 4096
L = 50
LP = 56
D = 64
DS = 16

NC, NS = 2, 16
NW = NC * NS
ROWS_PER_W = B * LP // NW
SUB = 128
K_INNER = 8
CHUNK = SUB * K_INNER
N_OUTER = ROWS_PER_W // CHUNK
N_IDX_ROWS = ROWS_PER_W // SUB
ACC_PER_W = B // NW


def _make_sc_gather(interpret=False):
    mesh = plsc.VectorSubcoreMesh(core_axis_name="c", subcore_axis_name="s",
                                  num_cores=NC, num_subcores=NS)

    @functools.partial(
        pl.kernel,
        out_type=(
            jax.ShapeDtypeStruct((B * LP, D), jnp.float32),
            jax.ShapeDtypeStruct((B, D), jnp.float32),
            jax.ShapeDtypeStruct((B, DS), jnp.float32),
            jax.ShapeDtypeStruct((B, DS), jnp.float32),
        ),
        mesh=mesh,
        scratch_types=[
            pltpu.VMEM((N_IDX_ROWS, SUB), jnp.int32),
            pltpu.VMEM((CHUNK, D), jnp.float32),
            pltpu.VMEM((1, ACC_PER_W), jnp.int32),
            pltpu.VMEM((ACC_PER_W, D), jnp.float32),
            pltpu.VMEM((1, ACC_PER_W), jnp.int32),
            pltpu.VMEM((ACC_PER_W, DS), jnp.float32),
            pltpu.VMEM((1, ACC_PER_W), jnp.int32),
            pltpu.VMEM((ACC_PER_W, DS), jnp.float32),
            pltpu.SemaphoreType.DMA,
            pltpu.SemaphoreType.DMA,
        ],
        compiler_params=pltpu.CompilerParams(use_tc_tiling_on_sc=False),
        interpret=interpret,
    )
    def sc_gather(chain_ids2d, acc_idx2d, hour_idx2d, wd_idx2d,
                  chain_table, account_table, hour_table, weekday_table,
                  x_out, acc_out, h_out, w_out,
                  cidx_v, rows_v, aidx_v, arows_v, hidx_v, hrows_v,
                  widx_v, wrows_v, sem, sem2):
        wid = lax.axis_index("s") * NC + lax.axis_index("c")

        pltpu.sync_copy(acc_idx2d.at[wid], aidx_v)
        pltpu.sync_copy(hour_idx2d.at[wid], hidx_v)
        pltpu.sync_copy(wd_idx2d.at[wid], widx_v)
        a_cp = pltpu.make_async_copy(account_table.at[aidx_v.at[0]], arows_v,
                                     sem2)
        h_cp = pltpu.make_async_copy(hour_table.at[hidx_v.at[0]], hrows_v,
                                     sem2)
        w_cp = pltpu.make_async_copy(weekday_table.at[widx_v.at[0]], wrows_v,
                                     sem2)
        a_cp.start()
        h_cp.start()
        w_cp.start()

        pltpu.sync_copy(chain_ids2d.at[wid], cidx_v)
        base = wid * ROWS_PER_W

        def outer(g, carry):
            cps = [
                pltpu.make_async_copy(
                    chain_table.at[cidx_v.at[g * K_INNER + j]],
                    rows_v.at[pl.ds(j * SUB, SUB)], sem)
                for j in range(K_INNER)
            ]
            for cp in cps:
                cp.start()
            for cp in cps:
                cp.wait()
            pltpu.sync_copy(rows_v, x_out.at[pl.ds(base + g * CHUNK, CHUNK)])
            return carry

        lax.fori_loop(0, N_OUTER, outer, 0)

        a_cp.wait()
        h_cp.wait()
        w_cp.wait()
        abase = wid * ACC_PER_W
        pltpu.sync_copy(arows_v, acc_out.at[pl.ds(abase, ACC_PER_W)])
        pltpu.sync_copy(hrows_v, h_out.at[pl.ds(abase, ACC_PER_W)])
        pltpu.sync_copy(wrows_v, w_out.at[pl.ds(abase, ACC_PER_W)])

    return sc_gather


BB = 128
PREC = lax.Precision.DEFAULT


def _attn_body(ids_ref, x_ref, acc_ref, h_ref, w_ref,
               wq_ref, bq_ref, wk_ref, bk_ref, wv_ref, bv_ref,
               wo_ref, bo_ref, lng_ref, lnb_ref, out_ref):
    x = x_ref[...]
    x2 = x.reshape(BB * LP, D)
    q = jnp.dot(x2, wq_ref[...], precision=PREC,
                preferred_element_type=jnp.float32) + bq_ref[...]
    k = jnp.dot(x2, wk_ref[...], precision=PREC,
                preferred_element_type=jnp.float32) + bk_ref[...]
    v = jnp.dot(x2, wv_ref[...], precision=PREC,
                preferred_element_type=jnp.float32) + bv_ref[...]
    q3 = q.reshape(BB, LP, D)
    k3 = k.reshape(BB, LP, D)
    v3 = v.reshape(BB, LP, D)
    scores = lax.dot_general(q3, k3, (((2,), (2,)), ((0,), (0,))),
                             precision=PREC,
                             preferred_element_type=jnp.float32) * 0.125
    idmask = ids_ref[...] != 0
    padcol = lax.broadcasted_iota(jnp.int32, (BB, LP), 1) >= L
    mask = idmask & jnp.logical_not(padcol)
    scores = jnp.where(idmask[:, None, :], scores, jnp.float32(-1e9))
    scores = jnp.where(padcol[:, None, :], jnp.float32(-2e9), scores)
    m = jnp.max(scores, axis=-1, keepdims=True)
    e = jnp.exp(scores - m)
    attn = e / jnp.sum(e, axis=-1, keepdims=True)
    ctx = lax.dot_general(attn, v3, (((2,), (1,)), ((0,), (0,))),
                          precision=PREC,
                          preferred_element_type=jnp.float32)
    ctx2 = ctx.reshape(BB * LP, D)
    y = x2 + jnp.dot(ctx2, wo_ref[...], precision=PREC,
                     preferred_element_type=jnp.float32) + bo_ref[...]
    mean = jnp.mean(y, axis=-1, keepdims=True)
    var = jnp.mean((y - mean) ** 2, axis=-1, keepdims=True)
    y = lng_ref[...] * (y - mean) / jnp.sqrt(var + 1e-3) + lnb_ref[...]
    y3 = y.reshape(BB, LP, D)
    mf = mask.astype(jnp.float32)[..., None]
    pooled = jnp.sum(y3 * mf, axis=1) / jnp.clip(jnp.sum(mf, axis=1), 1.0,
                                                 None)
    out_ref[:, 0:D] = acc_ref[...]
    out_ref[:, D:2 * D] = pooled
    out_ref[:, 2 * D:2 * D + DS] = h_ref[...]
    out_ref[:, 2 * D + DS:2 * D + 2 * DS] = w_ref[...]


def _make_tc_attn(interpret=False):
    full = lambda shape: pl.BlockSpec(shape, lambda i: (0,) * len(shape))
    return pl.pallas_call(
        _attn_body,
        grid=(B // BB,),
        in_specs=[
            pl.BlockSpec((BB, LP), lambda i: (i, 0)),
            pl.BlockSpec((BB, LP, D), lambda i: (i, 0, 0)),
            pl.BlockSpec((BB, D), lambda i: (i, 0)),
            pl.BlockSpec((BB, DS), lambda i: (i, 0)),
            pl.BlockSpec((BB, DS), lambda i: (i, 0)),
            full((D, D)), full((1, D)),
            full((D, D)), full((1, D)),
            full((D, D)), full((1, D)),
            full((D, D)), full((1, D)),
            full((1, D)), full((1, D)),
        ],
        out_specs=pl.BlockSpec((BB, 2 * D + 2 * DS), lambda i: (i, 0)),
        out_shape=jax.ShapeDtypeStruct((B, 2 * D + 2 * DS), jnp.float32),
        interpret=interpret,
    )


_tc_attn = _make_tc_attn()


@functools.cache
def _get_sc_gather():
    return _make_sc_gather()


def kernel(account_idx, prev_chain_ids, hour_idx, weekday_idx,
           account_table, chain_table, Wq, bq, Wk, bk, Wv, bv, Wo, bo,
           ln_gamma, ln_beta, hour_table, weekday_table):
    pad = (jax.lax.broadcasted_iota(jnp.int32, (B, LP - L), 0) * 7919 + 1
           ) % chain_table.shape[0]
    ids_pad = jnp.concatenate([prev_chain_ids, pad], axis=1)
    chain_ids2d = ids_pad.reshape(NW, N_IDX_ROWS, SUB)
    acc_idx2d = account_idx.reshape(NW, 1, ACC_PER_W)
    hour_idx2d = hour_idx.reshape(NW, 1, ACC_PER_W)
    wd_idx2d = weekday_idx.reshape(NW, 1, ACC_PER_W)
    xf, acc, h, w = _get_sc_gather()(chain_ids2d, acc_idx2d, hour_idx2d,
                                     wd_idx2d, chain_table, account_table,
                                     hour_table, weekday_table)
    return _tc_attn(ids_pad, xf.reshape(B, LP, D), acc, h, w,
                    Wq, bq.reshape(1, D), Wk, bk.reshape(1, D),
                    Wv, bv.reshape(1, D), Wo, bo.reshape(1, D),
                    ln_gamma.reshape(1, D), ln_beta.reshape(1, D))


# speedup vs baseline: 1.9727x; 1.0064x over previous
import functools

import jax
import jax.numpy as jnp
from jax import lax
from jax.experimental import pallas as pl
from jax.experimental.pallas import tpu as pltpu
from jax.experimental.pallas import tpu_sc as plsc

B = 4096
L = 50
LP = 56
D = 64
DS = 16

CH = 2
BC = B // CH

NC, NS = 2, 16
NW = NC * NS
SUB = 128
K_INNER = 7
CHUNK = SUB * K_INNER

ROWS_PER_W = BC * LP // NW
N_IDX_ROWS = ROWS_PER_W // SUB
N_OUTER = ROWS_PER_W // CHUNK
ACC_PER_W = BC // NW


def _make_sc_gather(interpret=False):
    mesh = plsc.VectorSubcoreMesh(core_axis_name="c", subcore_axis_name="s",
                                  num_cores=NC, num_subcores=NS)

    @functools.partial(
        pl.kernel,
        out_type=(
            jax.ShapeDtypeStruct((BC * LP, D), jnp.float32),
            jax.ShapeDtypeStruct((BC, D), jnp.float32),
            jax.ShapeDtypeStruct((BC, DS), jnp.float32),
            jax.ShapeDtypeStruct((BC, DS), jnp.float32),
        ),
        mesh=mesh,
        scratch_types=[
            pltpu.VMEM((N_IDX_ROWS, SUB), jnp.int32),
            pltpu.VMEM((CHUNK, D), jnp.float32),
            pltpu.VMEM((1, ACC_PER_W), jnp.int32),
            pltpu.VMEM((ACC_PER_W, D), jnp.float32),
            pltpu.VMEM((1, ACC_PER_W), jnp.int32),
            pltpu.VMEM((ACC_PER_W, DS), jnp.float32),
            pltpu.VMEM((1, ACC_PER_W), jnp.int32),
            pltpu.VMEM((ACC_PER_W, DS), jnp.float32),
            pltpu.SemaphoreType.DMA,
            pltpu.SemaphoreType.DMA,
        ],
        compiler_params=pltpu.CompilerParams(use_tc_tiling_on_sc=False),
        interpret=interpret,
    )
    def sc_gather(chain_ids3d, acc_idx2d, hour_idx2d, wd_idx2d,
                  chain_table, account_table, hour_table, weekday_table,
                  x_out, acc_out, h_out, w_out,
                  cidx_v, rows_v, aidx_v, arows_v, hidx_v, hrows_v,
                  widx_v, wrows_v, sem, sem2):
        wid = lax.axis_index("s") * NC + lax.axis_index("c")

        pltpu.sync_copy(acc_idx2d.at[wid], aidx_v)
        pltpu.sync_copy(hour_idx2d.at[wid], hidx_v)
        pltpu.sync_copy(wd_idx2d.at[wid], widx_v)
        a_cp = pltpu.make_async_copy(account_table.at[aidx_v.at[0]], arows_v,
                                     sem2)
        h_cp = pltpu.make_async_copy(hour_table.at[hidx_v.at[0]], hrows_v,
                                     sem2)
        w_cp = pltpu.make_async_copy(weekday_table.at[widx_v.at[0]], wrows_v,
                                     sem2)
        a_cp.start()
        h_cp.start()
        w_cp.start()

        pltpu.sync_copy(chain_ids3d.at[wid], cidx_v)
        base = wid * ROWS_PER_W

        def outer(g, carry):
            cps = [
                pltpu.make_async_copy(
                    chain_table.at[cidx_v.at[g * K_INNER + j]],
                    rows_v.at[pl.ds(j * SUB, SUB)], sem)
                for j in range(K_INNER)
            ]
            for cp in cps:
                cp.start()
            for cp in cps:
                cp.wait()
            pltpu.sync_copy(rows_v, x_out.at[pl.ds(base + g * CHUNK, CHUNK)])
            return carry

        lax.fori_loop(0, N_OUTER, outer, 0)

        a_cp.wait()
        h_cp.wait()
        w_cp.wait()
        abase = wid * ACC_PER_W
        pltpu.sync_copy(arows_v, acc_out.at[pl.ds(abase, ACC_PER_W)])
        pltpu.sync_copy(hrows_v, h_out.at[pl.ds(abase, ACC_PER_W)])
        pltpu.sync_copy(wrows_v, w_out.at[pl.ds(abase, ACC_PER_W)])

    return sc_gather


BB = 128
PREC = lax.Precision.DEFAULT


def _attn_body(ids_ref, x_ref, acc_ref, h_ref, w_ref,
               wq_ref, bq_ref, wk_ref, bk_ref, wv_ref, bv_ref,
               wo_ref, bo_ref, lng_ref, lnb_ref, out_ref):
    x = x_ref[...]
    x2 = x.reshape(BB * LP, D)
    q = jnp.dot(x2, wq_ref[...], precision=PREC,
                preferred_element_type=jnp.float32) + bq_ref[...]
    k = jnp.dot(x2, wk_ref[...], precision=PREC,
                preferred_element_type=jnp.float32) + bk_ref[...]
    v = jnp.dot(x2, wv_ref[...], precision=PREC,
                preferred_element_type=jnp.float32) + bv_ref[...]
    q3 = q.reshape(BB, LP, D)
    k3 = k.reshape(BB, LP, D)
    v3 = v.reshape(BB, LP, D)
    scores = lax.dot_general(q3, k3, (((2,), (2,)), ((0,), (0,))),
                             precision=PREC,
                             preferred_element_type=jnp.float32) * 0.125
    idmask = ids_ref[...] != 0
    padcol = lax.broadcasted_iota(jnp.int32, (BB, LP), 1) >= L
    mask = idmask & jnp.logical_not(padcol)
    scores = jnp.where(idmask[:, None, :], scores, jnp.float32(-1e9))
    scores = jnp.where(padcol[:, None, :], jnp.float32(-2e9), scores)
    m = jnp.max(scores, axis=-1, keepdims=True)
    e = jnp.exp(scores - m)
    attn = e / jnp.sum(e, axis=-1, keepdims=True)
    ctx = lax.dot_general(attn, v3, (((2,), (1,)), ((0,), (0,))),
                          precision=PREC,
                          preferred_element_type=jnp.float32)
    ctx2 = ctx.reshape(BB * LP, D)
    y = x2 + jnp.dot(ctx2, wo_ref[...], precision=PREC,
                     preferred_element_type=jnp.float32) + bo_ref[...]
    mean = jnp.mean(y, axis=-1, keepdims=True)
    var = jnp.mean((y - mean) ** 2, axis=-1, keepdims=True)
    y = lng_ref[...] * (y - mean) / jnp.sqrt(var + 1e-3) + lnb_ref[...]
    y3 = y.reshape(BB, LP, D)
    mf = mask.astype(jnp.float32)[..., None]
    pooled = jnp.sum(y3 * mf, axis=1) / jnp.clip(jnp.sum(mf, axis=1), 1.0,
                                                 None)
    out_ref[:, 0:D] = acc_ref[...]
    out_ref[:, D:2 * D] = pooled
    out_ref[:, 2 * D:2 * D + DS] = h_ref[...]
    out_ref[:, 2 * D + DS:2 * D + 2 * DS] = w_ref[...]


def _make_tc_attn(interpret=False):
    full = lambda shape: pl.BlockSpec(shape, lambda i: (0,) * len(shape))
    return pl.pallas_call(
        _attn_body,
        grid=(BC // BB,),
        in_specs=[
            pl.BlockSpec((BB, LP), lambda i: (i, 0)),
            pl.BlockSpec((BB, LP, D), lambda i: (i, 0, 0)),
            pl.BlockSpec((BB, D), lambda i: (i, 0)),
            pl.BlockSpec((BB, DS), lambda i: (i, 0)),
            pl.BlockSpec((BB, DS), lambda i: (i, 0)),
            full((D, D)), full((1, D)),
            full((D, D)), full((1, D)),
            full((D, D)), full((1, D)),
            full((D, D)), full((1, D)),
            full((1, D)), full((1, D)),
        ],
        out_specs=pl.BlockSpec((BB, 2 * D + 2 * DS), lambda i: (i, 0)),
        out_shape=jax.ShapeDtypeStruct((BC, 2 * D + 2 * DS), jnp.float32),
        interpret=interpret,
    )


_tc_attn = _make_tc_attn()


@functools.cache
def _get_sc_gather():
    return _make_sc_gather()


def kernel(account_idx, prev_chain_ids, hour_idx, weekday_idx,
           account_table, chain_table, Wq, bq, Wk, bk, Wv, bv, Wo, bo,
           ln_gamma, ln_beta, hour_table, weekday_table):
    pad = (lax.broadcasted_iota(jnp.int32, (B, LP - L), 0) * 7919 + 1
           ) % chain_table.shape[0]
    ids_pad = jnp.concatenate([prev_chain_ids, pad], axis=1)

    sc = _get_sc_gather()
    weights = (Wq, bq.reshape(1, D), Wk, bk.reshape(1, D),
               Wv, bv.reshape(1, D), Wo, bo.reshape(1, D),
               ln_gamma.reshape(1, D), ln_beta.reshape(1, D))
    outs = []
    for s in range(CH):
        sl = slice(s * BC, (s + 1) * BC)
        xf, acc, h, w = sc(ids_pad[sl].reshape(NW, N_IDX_ROWS, SUB),
                           account_idx[sl].reshape(NW, 1, ACC_PER_W),
                           hour_idx[sl].reshape(NW, 1, ACC_PER_W),
                           weekday_idx[sl].reshape(NW, 1, ACC_PER_W),
                           chain_table, account_table, hour_table,
                           weekday_table)
        outs.append(_tc_attn(ids_pad[sl], xf.reshape(BC, LP, D), acc, h, w,
                             *weights))
    return outs[0] if CH == 1 else jnp.concatenate(outs, axis=0)
